# baseline (speedup 1.0000x reference)
import functools

import jax
import jax.numpy as jnp
from jax import lax
from jax.experimental import pallas as pl
from jax.experimental.pallas import tpu as pltpu
from jax.experimental.pallas import tpu_sc as plsc

N = 10000
E = 320000
D = 128
H1 = 256
H2 = 128
C = 2

NC = 2
NS = 16
NW = NC * NS
NP = 10240
RPT = NP // NS
CW = 16
K = 125
KB = 128
CHUNKS = E // K
CPW = CHUNKS // NW
SECC = 8
SEC = CPW // SECC
NBUF = 2
NBUFC = 4
ZB = 80

BN_SCALE = 1.0 / (1.0 + 1e-5) ** 0.5


_SC_MESH = plsc.VectorSubcoreMesh(core_axis_name="c", subcore_axis_name="s")


def _fill_buf(buf, value):
  @pl.loop(0, KB)
  def _fill_rows(r):
    for j in range(D // 16):
      buf[r, pl.ds(j * 16, 16)] = jnp.full((16,), value, jnp.float32)


def _make_sc_segsum(with_cnt: bool):
  def body(val_hbm, src_hbm, dst_hbm, part_hbm, *rest):
    if with_cnt:
      cnt_hbm, acc_sh, src_l, dst_l, *bufs_and_sems = rest
    else:
      acc_sh, src_l, dst_l, *bufs_and_sems = rest
    rows = bufs_and_sems[:NBUF]
    semg = bufs_and_sems[NBUF:2 * NBUF]
    sems = bufs_and_sems[2 * NBUF:3 * NBUF]
    semi = bufs_and_sems[3 * NBUF]
    c = lax.axis_index("c")
    s = lax.axis_index("s")
    gwid = s * NC + c
    r0 = s * RPT
    lo = gwid * CPW

    _fill_buf(rows[0], 0.0)
    for t in range(RPT // ZB):
      pltpu.sync_copy(rows[0].at[pl.ds(0, ZB)],
                      acc_sh.at[pl.ds(r0 + t * ZB, ZB)])
    plsc.subcore_barrier()

    def run_section(src_ref, dst_ref):
      gath = {}
      scat = {}
      for j in range(SECC):
        b = j % NBUF
        if j >= NBUF:
          scat[j - NBUF].wait()
        gath[j] = pltpu.async_copy(val_hbm.at[src_ref.at[j]],
                                   rows[b].at[pl.ds(0, K)], semg[b])
        gath[j].wait()
        scat[j] = pltpu.async_copy(rows[b].at[pl.ds(0, K)],
                                   acc_sh.at[dst_ref.at[j]],
                                   sems[b], add=True)
      for j in range(SECC - NBUF, SECC):
        scat[j].wait()

    pltpu.sync_copy(src_hbm.at[pl.ds(lo, SECC)], src_l.at[0])
    pltpu.sync_copy(dst_hbm.at[pl.ds(lo, SECC)], dst_l.at[0])

    @pl.loop(0, SEC // 2)
    def _section_pair(p):
      b1 = lo + (2 * p + 1) * SECC
      la = pltpu.async_copy(src_hbm.at[pl.ds(b1, SECC)], src_l.at[1], semi)
      lb = pltpu.async_copy(dst_hbm.at[pl.ds(b1, SECC)], dst_l.at[1], semi)
      run_section(src_l.at[0], dst_l.at[0])
      la.wait()
      lb.wait()
      b2 = lo + jnp.minimum((2 * p + 2), SEC - 1) * SECC
      lc = pltpu.async_copy(src_hbm.at[pl.ds(b2, SECC)], src_l.at[0], semi)
      ld = pltpu.async_copy(dst_hbm.at[pl.ds(b2, SECC)], dst_l.at[0], semi)
      run_section(src_l.at[1], dst_l.at[1])
      lc.wait()
      ld.wait()

    plsc.subcore_barrier()
    o0 = c * NP + r0
    for t in range(RPT // KB):
      pltpu.sync_copy(acc_sh.at[pl.ds(r0 + t * KB, KB)], rows[0])
      pltpu.sync_copy(rows[0], part_hbm.at[pl.ds(o0 + t * KB, KB)])

    if with_cnt:
      plsc.subcore_barrier()
      _fill_buf(rows[0], 0.0)
      for t in range(RPT // ZB):
        pltpu.sync_copy(rows[0].at[pl.ds(0, ZB)],
                        acc_sh.at[pl.ds(r0 + t * ZB, ZB)])
      _fill_buf(rows[0], 1.0)
      plsc.subcore_barrier()

      sempool = list(semg) + list(sems)

      @pl.loop(0, SEC)
      def _cnt_section(sec):
        pltpu.sync_copy(dst_hbm.at[pl.ds(lo + sec * SECC, SECC)],
                        dst_l.at[0])
        ss = {}
        for j in range(SECC):
          if j >= NBUFC:
            ss[j - NBUFC].wait()
          ss[j] = pltpu.async_copy(rows[0].at[pl.ds(0, K)],
                                   acc_sh.at[dst_l.at[0].at[j]],
                                   sempool[j % NBUFC], add=True)
        for j in range(SECC - NBUFC, SECC):
          ss[j].wait()

      plsc.subcore_barrier()
      for t in range(RPT // KB):
        pltpu.sync_copy(acc_sh.at[pl.ds(r0 + t * KB, KB)], rows[1])
        pltpu.sync_copy(rows[1], cnt_hbm.at[pl.ds(o0 + t * KB, KB)])

  out_type = [jax.ShapeDtypeStruct((NC * NP, D), jnp.float32)]
  if with_cnt:
    out_type.append(jax.ShapeDtypeStruct((NC * NP, D), jnp.float32))
  return pl.kernel(
      body,
      out_type=tuple(out_type),
      mesh=_SC_MESH,
      scratch_types=[
          pltpu.VMEM_SHARED((NP, D), jnp.float32),
          pltpu.VMEM((2, SECC, K), jnp.int32),
          pltpu.VMEM((2, SECC, K), jnp.int32),
      ] + [pltpu.VMEM((KB, D), jnp.float32) for _ in range(NBUF)]
        + [pltpu.SemaphoreType.DMA for _ in range(2 * NBUF)]
        + [pltpu.SemaphoreType.DMA])


_sc_segsum_cnt = _make_sc_segsum(True)
_sc_segsum = _make_sc_segsum(False)

R = 1000
NB = N // R


def _tc1_body(x_ref, p0_ref, p1_ref, c0_ref, c1_ref, wl_ref, br_ref,
              wr_ref, gb_ref, wp_ref, bp_ref, wl2_ref, h_ref, hw_ref):
  cnt = c0_ref[:, 0:1] + c1_ref[:, 0:1]
  inv = 1.0 / jnp.maximum(cnt, 1.0)
  agg = (p0_ref[...] + p1_ref[...]) * inv
  x = x_ref[...]
  out = (jnp.dot(agg, wl_ref[...], preferred_element_type=jnp.float32)
         + jnp.dot(x, wr_ref[...], preferred_element_type=jnp.float32)
         + br_ref[0:1, :])
  nrm = jnp.sqrt(jnp.sum(out * out, axis=-1, keepdims=True))
  out = out / jnp.maximum(nrm, 1e-12)
  out = jnp.maximum(out * (BN_SCALE * gb_ref[0:1, :]) + gb_ref[1:2, :], 0.0)
  h = out + jnp.dot(x, wp_ref[...], preferred_element_type=jnp.float32) \
      + bp_ref[0:1, :]
  h_ref[...] = h
  hw_ref[...] = jnp.dot(h, wl2_ref[...], preferred_element_type=jnp.float32)


def _tc1(x, p0, p1, c0, c1, wl1t, br1, wr1t, gb1, wp1t, bp1, wl2t):
  row = lambda i: (i, 0)
  full = lambda i: (0, 0)
  return pl.pallas_call(
      _tc1_body,
      grid=(NB,),
      in_specs=[
          pl.BlockSpec((R, D), row),
          pl.BlockSpec((R, D), row),
          pl.BlockSpec((R, D), row),
          pl.BlockSpec((R, CW), row),
          pl.BlockSpec((R, CW), row),
          pl.BlockSpec((D, H1), full),
          pl.BlockSpec((1, H1), full),
          pl.BlockSpec((D, H1), full),
          pl.BlockSpec((2, H1), full),
          pl.BlockSpec((D, H1), full),
          pl.BlockSpec((1, H1), full),
          pl.BlockSpec((H1, H2), full),
      ],
      out_specs=[
          pl.BlockSpec((R, H1), row),
          pl.BlockSpec((R, H2), row),
      ],
      out_shape=[
          jax.ShapeDtypeStruct((N, H1), jnp.float32),
          jax.ShapeDtypeStruct((N, H2), jnp.float32),
      ],
  )(x, p0, p1, c0, c1, wl1t, br1, wr1t, gb1, wp1t, bp1, wl2t)


def _tc2_body(h_ref, p0_ref, p1_ref, c0_ref, c1_ref, bl2_ref, wr2_ref,
              gb2_ref, wp2_ref, bp2_ref, wc1_ref, bc1_ref, gbc1_ref,
              wc2_ref, bc2_ref, gbc2_ref, wc3_ref, bc3_ref,
              logits_ref, emb_ref):
  i = pl.program_id(0)
  cnt = c0_ref[:, 0:1] + c1_ref[:, 0:1]
  inv = 1.0 / jnp.maximum(cnt, 1.0)
  h = h_ref[...]
  out = ((p0_ref[...] + p1_ref[...]) * inv + bl2_ref[0:1, :]
         + jnp.dot(h, wr2_ref[...], preferred_element_type=jnp.float32))
  nrm = jnp.sqrt(jnp.sum(out * out, axis=-1, keepdims=True))
  out = out / jnp.maximum(nrm, 1e-12)
  out = jnp.maximum(out * (BN_SCALE * gb2_ref[0:1, :]) + gb2_ref[1:2, :], 0.0)
  h2 = out + jnp.dot(h, wp2_ref[...], preferred_element_type=jnp.float32) \
      + bp2_ref[0:1, :]
  psum = jnp.sum(h2, axis=0, keepdims=True)

  @pl.when(i == 0)
  def _():
    emb_ref[...] = psum

  @pl.when(i > 0)
  def _():
    emb_ref[...] = emb_ref[...] + psum

  @pl.when(i == NB - 1)
  def _():
    emb = emb_ref[...] * (1.0 / N)
    emb_ref[...] = emb
    z = jnp.dot(emb, wc1_ref[...], preferred_element_type=jnp.float32) \
        + bc1_ref[0:1, :]
    z = jnp.maximum(z * (BN_SCALE * gbc1_ref[0:1, :]) + gbc1_ref[1:2, :], 0.0)
    z = jnp.dot(z, wc2_ref[...], preferred_element_type=jnp.float32) \
        + bc2_ref[0:1, :]
    z = jnp.maximum(z * (BN_SCALE * gbc2_ref[0:1, :]) + gbc2_ref[1:2, :], 0.0)
    logits_ref[...] = jnp.dot(z, wc3_ref[...],
                              preferred_element_type=jnp.float32) \
        + bc3_ref[0:1, :]


def _tc2(h, p0, p1, c0, c1, bl2, wr2t, gb2, wp2t, bp2,
         wc1t, bc1, gbc1, wc2t, bc2, gbc2, wc3t, bc3):
  row = lambda i: (i, 0)
  full = lambda i: (0, 0)
  return pl.pallas_call(
      _tc2_body,
      grid=(NB,),
      in_specs=[
          pl.BlockSpec((R, H1), row),
          pl.BlockSpec((R, H2), row),
          pl.BlockSpec((R, H2), row),
          pl.BlockSpec((R, CW), row),
          pl.BlockSpec((R, CW), row),
          pl.BlockSpec((1, H2), full),
          pl.BlockSpec((H1, H2), full),
          pl.BlockSpec((2, H2), full),
          pl.BlockSpec((H1, H2), full),
          pl.BlockSpec((1, H2), full),
          pl.BlockSpec((H2, 256), full),
          pl.BlockSpec((1, 256), full),
          pl.BlockSpec((2, 256), full),
          pl.BlockSpec((256, 128), full),
          pl.BlockSpec((1, 128), full),
          pl.BlockSpec((2, 128), full),
          pl.BlockSpec((128, C), full),
          pl.BlockSpec((1, C), full),
      ],
      out_specs=[
          pl.BlockSpec((1, C), full),
          pl.BlockSpec((1, H2), full),
      ],
      out_shape=[
          jax.ShapeDtypeStruct((1, C), jnp.float32),
          jax.ShapeDtypeStruct((1, H2), jnp.float32),
      ],
      compiler_params=pltpu.CompilerParams(
          dimension_semantics=("arbitrary",)),
  )(h, p0, p1, c0, c1, bl2, wr2t, gb2, wp2t, bp2,
    wc1t, bc1, gbc1, wc2t, bc2, gbc2, wc3t, bc3)


def kernel(x, edge_index, W_l1, b_l1, W_r1, g1, be1, W_p1, b_p1, W_l2, b_l2,
           W_r2, g2, be2, W_p2, b_p2, W_c1, b_c1, g_c1, be_c1, W_c2, b_c2,
           g_c2, be_c2, W_c3, b_c3):
  src = edge_index[0].reshape(CHUNKS, K)
  dst = edge_index[1].reshape(CHUNKS, K)

  part1, cntp = _sc_segsum_cnt(x, src, dst)
  p0, p1 = part1[:N], part1[NP:NP + N]
  c0, c1 = cntp[:N, :CW], cntp[NP:NP + N, :CW]

  gb1 = jnp.stack([g1, be1])
  h, hw = _tc1(x, p0, p1, c0, c1,
               W_l1.T, b_l1[None, :], W_r1.T, gb1, W_p1.T, b_p1[None, :],
               W_l2.T)

  (part2,) = _sc_segsum(hw, src, dst)
  q0, q1 = part2[:N], part2[NP:NP + N]

  gb2 = jnp.stack([g2, be2])
  gbc1 = jnp.stack([g_c1, be_c1])
  gbc2 = jnp.stack([g_c2, be_c2])
  logits, emb = _tc2(h, q0, q1, c0, c1,
                     b_l2[None, :], W_r2.T, gb2, W_p2.T, b_p2[None, :],
                     W_c1.T, b_c1[None, :], gbc1,
                     W_c2.T, b_c2[None, :], gbc2,
                     W_c3.T, b_c3[None, :])
  return (logits, emb)


# speedup vs baseline: 1.0094x; 1.0094x over previous
import functools

import jax
import jax.numpy as jnp
from jax import lax
from jax.experimental import pallas as pl
from jax.experimental.pallas import tpu as pltpu
from jax.experimental.pallas import tpu_sc as plsc

N = 10000
E = 320000
D = 128
H1 = 256
H2 = 128
C = 2

NC = 2
NS = 16
NW = NC * NS
NP = 10240
RPT = NP // NS
CW = 16
K = 125
KB = 128
CHUNKS = E // K
CPW = CHUNKS // NW
SECC = 8
SEC = CPW // SECC
NBUF = 2
NBUFC = 4
ZB = 80

BN_SCALE = 1.0 / (1.0 + 1e-5) ** 0.5


_SC_MESH = plsc.VectorSubcoreMesh(core_axis_name="c", subcore_axis_name="s")


def _fill_buf(buf, value):
  @pl.loop(0, KB)
  def _fill_rows(r):
    for j in range(D // 16):
      buf[r, pl.ds(j * 16, 16)] = jnp.full((16,), value, jnp.float32)


def _make_sc_segsum(with_cnt: bool):
  def body(val_hbm, src_hbm, dst_hbm, part_hbm, *rest):
    if with_cnt:
      cnt_hbm, acc_sh, src_l, dst_l, *bufs_and_sems = rest
    else:
      acc_sh, src_l, dst_l, *bufs_and_sems = rest
    rows = bufs_and_sems[:NBUF]
    semg = bufs_and_sems[NBUF:2 * NBUF]
    sems = bufs_and_sems[2 * NBUF:3 * NBUF]
    semi = bufs_and_sems[3 * NBUF]
    c = lax.axis_index("c")
    s = lax.axis_index("s")
    gwid = s * NC + c
    r0 = s * RPT
    lo = gwid * CPW

    _fill_buf(rows[0], 0.0)
    for t in range(RPT // ZB):
      pltpu.sync_copy(rows[0].at[pl.ds(0, ZB)],
                      acc_sh.at[pl.ds(r0 + t * ZB, ZB)])
    plsc.subcore_barrier()

    def run_section(src_ref, dst_ref):
      gath = {}
      scat = {}
      for j in range(SECC):
        b = j % NBUF
        if j >= NBUF:
          scat[j - NBUF].wait()
        gath[j] = pltpu.async_copy(val_hbm.at[src_ref.at[j]],
                                   rows[b].at[pl.ds(0, K)], semg[b])
        gath[j].wait()
        scat[j] = pltpu.async_copy(rows[b].at[pl.ds(0, K)],
                                   acc_sh.at[dst_ref.at[j]],
                                   sems[b], add=True)
      for j in range(SECC - NBUF, SECC):
        scat[j].wait()

    pltpu.sync_copy(src_hbm.at[pl.ds(lo, SECC)], src_l.at[0])
    pltpu.sync_copy(dst_hbm.at[pl.ds(lo, SECC)], dst_l.at[0])

    @pl.loop(0, SEC // 2)
    def _section_pair(p):
      b1 = lo + (2 * p + 1) * SECC
      la = pltpu.async_copy(src_hbm.at[pl.ds(b1, SECC)], src_l.at[1], semi)
      lb = pltpu.async_copy(dst_hbm.at[pl.ds(b1, SECC)], dst_l.at[1], semi)
      run_section(src_l.at[0], dst_l.at[0])
      la.wait()
      lb.wait()
      b2 = lo + jnp.minimum((2 * p + 2), SEC - 1) * SECC
      lc = pltpu.async_copy(src_hbm.at[pl.ds(b2, SECC)], src_l.at[0], semi)
      ld = pltpu.async_copy(dst_hbm.at[pl.ds(b2, SECC)], dst_l.at[0], semi)
      run_section(src_l.at[1], dst_l.at[1])
      lc.wait()
      ld.wait()

    plsc.subcore_barrier()
    o0 = c * NP + r0
    for t in range(RPT // KB):
      pltpu.sync_copy(acc_sh.at[pl.ds(r0 + t * KB, KB)], rows[0])
      pltpu.sync_copy(rows[0], part_hbm.at[pl.ds(o0 + t * KB, KB)])

    if with_cnt:
      plsc.subcore_barrier()
      _fill_buf(rows[0], 0.0)
      for t in range(RPT // ZB):
        pltpu.sync_copy(rows[0].at[pl.ds(0, ZB)],
                        acc_sh.at[pl.ds(r0 + t * ZB, ZB)])
      _fill_buf(rows[0], 1.0)
      plsc.subcore_barrier()

      sempool = list(semg) + list(sems)

      @pl.loop(0, SEC)
      def _cnt_section(sec):
        pltpu.sync_copy(dst_hbm.at[pl.ds(lo + sec * SECC, SECC)],
                        dst_l.at[0])
        ss = {}
        for j in range(SECC):
          if j >= NBUFC:
            ss[j - NBUFC].wait()
          ss[j] = pltpu.async_copy(rows[0].at[pl.ds(0, K)],
                                   acc_sh.at[dst_l.at[0].at[j]],
                                   sempool[j % NBUFC], add=True)
        for j in range(SECC - NBUFC, SECC):
          ss[j].wait()

      plsc.subcore_barrier()
      for t in range(RPT // KB):
        pltpu.sync_copy(acc_sh.at[pl.ds(r0 + t * KB, KB)], rows[1])
        pltpu.sync_copy(rows[1], cnt_hbm.at[pl.ds(o0 + t * KB, KB)])

  out_type = [jax.ShapeDtypeStruct((NC * NP, D), jnp.float32)]
  if with_cnt:
    out_type.append(jax.ShapeDtypeStruct((NC * NP, D), jnp.float32))
  return pl.kernel(
      body,
      out_type=tuple(out_type),
      mesh=_SC_MESH,
      scratch_types=[
          pltpu.VMEM_SHARED((NP, D), jnp.float32),
          pltpu.VMEM((2, SECC, K), jnp.int32),
          pltpu.VMEM((2, SECC, K), jnp.int32),
      ] + [pltpu.VMEM((KB, D), jnp.float32) for _ in range(NBUF)]
        + [pltpu.SemaphoreType.DMA for _ in range(2 * NBUF)]
        + [pltpu.SemaphoreType.DMA])


def _make_sc_count():
  def body(dst_hbm, cnt_hbm, cnt_sh, dst_l, ones_v, *sems):
    c = lax.axis_index("c")
    s = lax.axis_index("s")
    gwid = s * NC + c
    r0 = s * RPT
    lo = gwid * CPW

    pltpu.sync_copy(dst_hbm.at[pl.ds(lo, CPW)], dst_l)
    _fill_buf(ones_v, 0.0)
    for t in range(RPT // ZB):
      pltpu.sync_copy(ones_v.at[pl.ds(0, ZB)],
                      cnt_sh.at[pl.ds(r0 + t * ZB, ZB)])
    _fill_buf(ones_v, 1.0)
    plsc.subcore_barrier()

    @pl.loop(0, CPW // NBUFC)
    def _edge_group(g):
      j0 = g * NBUFC
      ss = []
      for b in range(NBUFC):
        ss.append(pltpu.async_copy(ones_v.at[pl.ds(0, K)],
                                   cnt_sh.at[dst_l.at[j0 + b]], sems[b],
                                   add=True))
      for b in range(NBUFC):
        ss[b].wait()

    plsc.subcore_barrier()
    o0 = c * NP + r0
    for t in range(RPT // KB):
      pltpu.sync_copy(cnt_sh.at[pl.ds(r0 + t * KB, KB)], ones_v)
      pltpu.sync_copy(ones_v, cnt_hbm.at[pl.ds(o0 + t * KB, KB)])

  return pl.kernel(
      body,
      out_type=(jax.ShapeDtypeStruct((NC * NP, D), jnp.float32),),
      mesh=_SC_MESH,
      scratch_types=[
          pltpu.VMEM_SHARED((NP, D), jnp.float32),
          pltpu.VMEM((CPW, K), jnp.int32),
          pltpu.VMEM((KB, D), jnp.float32),
      ] + [pltpu.SemaphoreType.DMA for _ in range(NBUFC)])


_sc_segsum = _make_sc_segsum(False)
_sc_count = _make_sc_count()

R = 1000
NB = N // R


def _tc1_body(x_ref, p0_ref, p1_ref, c0_ref, c1_ref, wl_ref, br_ref,
              wr_ref, gb_ref, wp_ref, bp_ref, wl2_ref, h_ref, hw_ref):
  cnt = c0_ref[:, 0:1] + c1_ref[:, 0:1]
  inv = 1.0 / jnp.maximum(cnt, 1.0)
  agg = (p0_ref[...] + p1_ref[...]) * inv
  x = x_ref[...]
  out = (jnp.dot(agg, wl_ref[...], preferred_element_type=jnp.float32)
         + jnp.dot(x, wr_ref[...], preferred_element_type=jnp.float32)
         + br_ref[0:1, :])
  nrm = jnp.sqrt(jnp.sum(out * out, axis=-1, keepdims=True))
  out = out / jnp.maximum(nrm, 1e-12)
  out = jnp.maximum(out * (BN_SCALE * gb_ref[0:1, :]) + gb_ref[1:2, :], 0.0)
  h = out + jnp.dot(x, wp_ref[...], preferred_element_type=jnp.float32) \
      + bp_ref[0:1, :]
  h_ref[...] = h
  hw_ref[...] = jnp.dot(h, wl2_ref[...], preferred_element_type=jnp.float32)


def _tc1(x, p0, p1, c0, c1, wl1t, br1, wr1t, gb1, wp1t, bp1, wl2t):
  row = lambda i: (i, 0)
  full = lambda i: (0, 0)
  return pl.pallas_call(
      _tc1_body,
      grid=(NB,),
      in_specs=[
          pl.BlockSpec((R, D), row),
          pl.BlockSpec((R, D), row),
          pl.BlockSpec((R, D), row),
          pl.BlockSpec((R, CW), row),
          pl.BlockSpec((R, CW), row),
          pl.BlockSpec((D, H1), full),
          pl.BlockSpec((1, H1), full),
          pl.BlockSpec((D, H1), full),
          pl.BlockSpec((2, H1), full),
          pl.BlockSpec((D, H1), full),
          pl.BlockSpec((1, H1), full),
          pl.BlockSpec((H1, H2), full),
      ],
      out_specs=[
          pl.BlockSpec((R, H1), row),
          pl.BlockSpec((R, H2), row),
      ],
      out_shape=[
          jax.ShapeDtypeStruct((N, H1), jnp.float32),
          jax.ShapeDtypeStruct((N, H2), jnp.float32),
      ],
  )(x, p0, p1, c0, c1, wl1t, br1, wr1t, gb1, wp1t, bp1, wl2t)


def _tc2_body(h_ref, p0_ref, p1_ref, c0_ref, c1_ref, bl2_ref, wr2_ref,
              gb2_ref, wp2_ref, bp2_ref, wc1_ref, bc1_ref, gbc1_ref,
              wc2_ref, bc2_ref, gbc2_ref, wc3_ref, bc3_ref,
              logits_ref, emb_ref):
  i = pl.program_id(0)
  cnt = c0_ref[:, 0:1] + c1_ref[:, 0:1]
  inv = 1.0 / jnp.maximum(cnt, 1.0)
  h = h_ref[...]
  out = ((p0_ref[...] + p1_ref[...]) * inv + bl2_ref[0:1, :]
         + jnp.dot(h, wr2_ref[...], preferred_element_type=jnp.float32))
  nrm = jnp.sqrt(jnp.sum(out * out, axis=-1, keepdims=True))
  out = out / jnp.maximum(nrm, 1e-12)
  out = jnp.maximum(out * (BN_SCALE * gb2_ref[0:1, :]) + gb2_ref[1:2, :], 0.0)
  h2 = out + jnp.dot(h, wp2_ref[...], preferred_element_type=jnp.float32) \
      + bp2_ref[0:1, :]
  psum = jnp.sum(h2, axis=0, keepdims=True)

  @pl.when(i == 0)
  def _():
    emb_ref[...] = psum

  @pl.when(i > 0)
  def _():
    emb_ref[...] = emb_ref[...] + psum

  @pl.when(i == NB - 1)
  def _():
    emb = emb_ref[...] * (1.0 / N)
    emb_ref[...] = emb
    z = jnp.dot(emb, wc1_ref[...], preferred_element_type=jnp.float32) \
        + bc1_ref[0:1, :]
    z = jnp.maximum(z * (BN_SCALE * gbc1_ref[0:1, :]) + gbc1_ref[1:2, :], 0.0)
    z = jnp.dot(z, wc2_ref[...], preferred_element_type=jnp.float32) \
        + bc2_ref[0:1, :]
    z = jnp.maximum(z * (BN_SCALE * gbc2_ref[0:1, :]) + gbc2_ref[1:2, :], 0.0)
    logits_ref[...] = jnp.dot(z, wc3_ref[...],
                              preferred_element_type=jnp.float32) \
        + bc3_ref[0:1, :]


def _tc2(h, p0, p1, c0, c1, bl2, wr2t, gb2, wp2t, bp2,
         wc1t, bc1, gbc1, wc2t, bc2, gbc2, wc3t, bc3):
  row = lambda i: (i, 0)
  full = lambda i: (0, 0)
  return pl.pallas_call(
      _tc2_body,
      grid=(NB,),
      in_specs=[
          pl.BlockSpec((R, H1), row),
          pl.BlockSpec((R, H2), row),
          pl.BlockSpec((R, H2), row),
          pl.BlockSpec((R, CW), row),
          pl.BlockSpec((R, CW), row),
          pl.BlockSpec((1, H2), full),
          pl.BlockSpec((H1, H2), full),
          pl.BlockSpec((2, H2), full),
          pl.BlockSpec((H1, H2), full),
          pl.BlockSpec((1, H2), full),
          pl.BlockSpec((H2, 256), full),
          pl.BlockSpec((1, 256), full),
          pl.BlockSpec((2, 256), full),
          pl.BlockSpec((256, 128), full),
          pl.BlockSpec((1, 128), full),
          pl.BlockSpec((2, 128), full),
          pl.BlockSpec((128, C), full),
          pl.BlockSpec((1, C), full),
      ],
      out_specs=[
          pl.BlockSpec((1, C), full),
          pl.BlockSpec((1, H2), full),
      ],
      out_shape=[
          jax.ShapeDtypeStruct((1, C), jnp.float32),
          jax.ShapeDtypeStruct((1, H2), jnp.float32),
      ],
      compiler_params=pltpu.CompilerParams(
          dimension_semantics=("arbitrary",)),
  )(h, p0, p1, c0, c1, bl2, wr2t, gb2, wp2t, bp2,
    wc1t, bc1, gbc1, wc2t, bc2, gbc2, wc3t, bc3)


def kernel(x, edge_index, W_l1, b_l1, W_r1, g1, be1, W_p1, b_p1, W_l2, b_l2,
           W_r2, g2, be2, W_p2, b_p2, W_c1, b_c1, g_c1, be_c1, W_c2, b_c2,
           g_c2, be_c2, W_c3, b_c3):
  src = edge_index[0].reshape(CHUNKS, K)
  dst = edge_index[1].reshape(CHUNKS, K)

  (part1,) = _sc_segsum(x, src, dst)
  (cntp,) = _sc_count(dst)
  p0, p1 = part1[:N], part1[NP:NP + N]
  c0, c1 = cntp[:N, :CW], cntp[NP:NP + N, :CW]

  gb1 = jnp.stack([g1, be1])
  h, hw = _tc1(x, p0, p1, c0, c1,
               W_l1.T, b_l1[None, :], W_r1.T, gb1, W_p1.T, b_p1[None, :],
               W_l2.T)

  (part2,) = _sc_segsum(hw, src, dst)
  q0, q1 = part2[:N], part2[NP:NP + N]

  gb2 = jnp.stack([g2, be2])
  gbc1 = jnp.stack([g_c1, be_c1])
  gbc2 = jnp.stack([g_c2, be_c2])
  logits, emb = _tc2(h, q0, q1, c0, c1,
                     b_l2[None, :], W_r2.T, gb2, W_p2.T, b_p2[None, :],
                     W_c1.T, b_c1[None, :], gbc1,
                     W_c2.T, b_c2[None, :], gbc2,
                     W_c3.T, b_c3[None, :])
  return (logits, emb)
